# baseline (speedup 1.0000x reference)
import functools
import math

import jax
import jax.numpy as jnp
from jax import lax
from jax.experimental import pallas as pl
from jax.experimental.pallas import tpu as pltpu
from jax.experimental.pallas import tpu_sc as plsc

VOCAB = 1000000
DIM = 64
BATCH = 16384
HIST = 50
SCALE = math.sqrt(DIM)

_info = plsc.get_sparse_core_info()
NC = _info.num_cores
NS = _info.num_subcores
NW = NC * NS
B_PER_W = BATCH // NW
JCOLS = BATCH // 128
J_PER_W = JCOLS // NW
N_UNITS = HIST * J_PER_W

_mesh = plsc.VectorSubcoreMesh(core_axis_name="c", subcore_axis_name="s")


@functools.partial(
    pl.kernel,
    mesh=_mesh,
    out_type=jax.ShapeDtypeStruct((HIST, DIM // 8, JCOLS, 1024), jnp.float32),
    scratch_types=[
        pltpu.VMEM((HIST, B_PER_W), jnp.int32),
        pltpu.VMEM((128, DIM), jnp.float32),
        pltpu.VMEM((128, DIM), jnp.float32),
        pltpu.VMEM((DIM // 8, 1024), jnp.float32),
        pltpu.VMEM((DIM // 8, 1024), jnp.float32),
        pltpu.SemaphoreType.DMA,
        pltpu.SemaphoreType.DMA,
        pltpu.SemaphoreType.DMA,
        pltpu.SemaphoreType.DMA,
    ],
    compiler_params=pltpu.CompilerParams(
        use_tc_tiling_on_sc=False, needs_layout_passes=False
    ),
)
def _embed_sc(idx_hbm, table_hbm, out_hbm, idx_v, rows0, rows1, patch0,
              patch1, g0, g1, s0, s1):
    wid = lax.axis_index("s") * NC + lax.axis_index("c")
    base_b = wid * B_PER_W
    jg_base = wid * J_PER_W
    rows = (rows0, rows1)
    patches = (patch0, patch1)
    gsems = (g0, g1)
    osems = (s0, s1)

    pltpu.sync_copy(idx_hbm.at[:, pl.ds(base_b, B_PER_W)], idx_v)

    lane = lax.iota(jnp.int32, 16)
    i_base = lane >> 3
    col_base = (lane & 7) * 128

    def unit_hj(u):
        return u // J_PER_W, u % J_PER_W

    def start_gather(u, b):
        h, jl = unit_hj(u)
        pltpu.async_copy(
            table_hbm.at[idx_v.at[h, pl.ds(jl * 128, 128)]], rows[b], gsems[b]
        )

    def scatter_scale(b):
        rbuf = rows[b]
        pbuf = patches[b]

        def row_body(r, carry):
            inner = col_base + r
            for k in range(DIM // 16):
                v = rbuf[r, pl.ds(16 * k, 16)] * SCALE
                plsc.store_scatter(pbuf, [i_base + 2 * k, inner], v)
            return carry

        lax.fori_loop(0, 128, row_body, 0, unroll=2)

    def out_slice(u):
        h, jl = unit_hj(u)
        return out_hbm.at[h, :, jg_base + jl]

    start_gather(0, 0)

    def pair_body(t, carry):
        for b in range(2):
            u = t * 2 + b

            @pl.when(u + 1 < N_UNITS)
            def _():
                start_gather(u + 1, 1 - b)

            pltpu.make_async_copy(
                table_hbm.at[idx_v.at[0, pl.ds(0, 128)]], rows[b], gsems[b]
            ).wait()

            @pl.when(u >= 2)
            def _():
                pltpu.make_async_copy(patches[b], out_slice(u), osems[b]).wait()

            scatter_scale(b)
            pltpu.async_copy(patches[b], out_slice(u), osems[b])
        return carry

    lax.fori_loop(0, N_UNITS // 2, pair_body, 0)

    pltpu.make_async_copy(patches[0], out_slice(0), osems[0]).wait()
    pltpu.make_async_copy(patches[1], out_slice(1), osems[1]).wait()


def kernel(input, table):
    out4 = _embed_sc(input.T, table)
    o5 = out4.reshape(HIST, DIM // 8, JCOLS, 8, 128)
    o6 = o5.transpose(2, 4, 0, 1, 3)
    return o6.reshape(BATCH, HIST, DIM)


# speedup vs baseline: 1.3179x; 1.3179x over previous
import functools
import math

import jax
import jax.numpy as jnp
from jax import lax
from jax.experimental import pallas as pl
from jax.experimental.pallas import tpu as pltpu
from jax.experimental.pallas import tpu_sc as plsc

VOCAB = 1000000
DIM = 64
BATCH = 16384
HIST = 50
SCALE = math.sqrt(DIM)

_info = plsc.get_sparse_core_info()
NC = _info.num_cores
NS = _info.num_subcores
NW = NC * NS
B_PER_W = BATCH // NW
JCOLS = BATCH // 128
J_PER_W = JCOLS // NW
N_UNITS = HIST * J_PER_W

_mesh = plsc.VectorSubcoreMesh(core_axis_name="c", subcore_axis_name="s")


@functools.partial(
    pl.kernel,
    mesh=_mesh,
    out_type=jax.ShapeDtypeStruct((HIST, DIM // 8, JCOLS, 1024), jnp.float32),
    scratch_types=[
        pltpu.VMEM((HIST, B_PER_W), jnp.int32),
        pltpu.VMEM((128, DIM), jnp.float32),
        pltpu.VMEM((128, DIM), jnp.float32),
        pltpu.VMEM((DIM // 8, 1024), jnp.float32),
        pltpu.VMEM((DIM // 8, 1024), jnp.float32),
        pltpu.SemaphoreType.DMA,
        pltpu.SemaphoreType.DMA,
        pltpu.SemaphoreType.DMA,
        pltpu.SemaphoreType.DMA,
    ],
    compiler_params=pltpu.CompilerParams(
        use_tc_tiling_on_sc=False, needs_layout_passes=False
    ),
)
def _embed_sc(idx_hbm, table_hbm, out_hbm, idx_v, rows0, rows1, patch0,
              patch1, g0, g1, s0, s1):
    wid = lax.axis_index("s") * NC + lax.axis_index("c")
    base_b = wid * B_PER_W
    jg_base = wid * J_PER_W
    rows = (rows0, rows1)
    patches = (patch0, patch1)
    gsems = (g0, g1)
    osems = (s0, s1)

    pltpu.sync_copy(idx_hbm.at[:, pl.ds(base_b, B_PER_W)], idx_v)

    lane = lax.iota(jnp.int32, 16)
    i_base = lane >> 3
    col_base = (lane & 7) * 128

    def unit_hj(u):
        return u // J_PER_W, u % J_PER_W

    def start_gather(u, b):
        h, jl = unit_hj(u)
        pltpu.async_copy(
            table_hbm.at[idx_v.at[h, pl.ds(jl * 128, 128)]], rows[b], gsems[b]
        )

    def scatter_scale(b):
        rbuf = rows[b]
        pbuf = patches[b]

        @plsc.parallel_loop(0, 128, step=1, unroll=8)
        def _(r):
            inner = col_base + r
            for k in range(DIM // 16):
                v = rbuf[r, pl.ds(16 * k, 16)] * SCALE
                plsc.store_scatter(pbuf, [i_base + 2 * k, inner], v)

    def out_slice(u):
        h, jl = unit_hj(u)
        return out_hbm.at[h, :, jg_base + jl]

    start_gather(0, 0)

    def pair_body(t, carry):
        for b in range(2):
            u = t * 2 + b

            @pl.when(u + 1 < N_UNITS)
            def _():
                start_gather(u + 1, 1 - b)

            pltpu.make_async_copy(
                table_hbm.at[idx_v.at[0, pl.ds(0, 128)]], rows[b], gsems[b]
            ).wait()

            @pl.when(u >= 2)
            def _():
                pltpu.make_async_copy(patches[b], out_slice(u), osems[b]).wait()

            scatter_scale(b)
            pltpu.async_copy(patches[b], out_slice(u), osems[b])
        return carry

    lax.fori_loop(0, N_UNITS // 2, pair_body, 0)

    pltpu.make_async_copy(patches[0], out_slice(0), osems[0]).wait()
    pltpu.make_async_copy(patches[1], out_slice(1), osems[1]).wait()


def kernel(input, table):
    out4 = _embed_sc(input.T, table)
    o5 = out4.reshape(HIST, DIM // 8, JCOLS, 8, 128)
    o6 = o5.transpose(2, 4, 0, 1, 3)
    return o6.reshape(BATCH, HIST, DIM)
